# Initial kernel scaffold; baseline (speedup 1.0000x reference)
import jax
import jax.numpy as jnp
from jax.experimental import pallas as pl


def kernel(x, edge_index, phi0_W, phi0_b, rho0_W, rho0_b, phi1_W, phi1_b, rho1_W, rho1_b, phi2_W, phi2_b, rho2_W, rho2_b, ro_phi_W, ro_phi_b, ro_rho_W, ro_rho_b):
    raise NotImplementedError("write your pallas kernel here")


# speedup vs baseline: 4.8825x; 4.8825x over previous
import functools

import jax
import jax.numpy as jnp
from jax import lax
from jax.experimental import pallas as pl
from jax.experimental.pallas import tpu as pltpu
from jax.experimental.pallas import tpu_sc as plsc

N = 10000
D = 256
HH = 128
E = 160000
NC, NS, LANES = 2, 16, 16
N_PAD = 10240
E_PAD = 163840
B = 128
WAVES_A = E_PAD // (NS * B)
WAVES_B = E_PAD // (NC * NS * B)
RT = N_PAD // NS
BLK = 2000


def _relu(v):
    return jnp.maximum(v, 0.0)


def _phi_body(x_ref, w_ref, b_ref, h0_ref, h1_ref, h2_ref):
    h = _relu(jnp.dot(x_ref[...], w_ref[...],
                      preferred_element_type=jnp.float32) + b_ref[...])
    h0_ref[...] = h[:, :HH]
    h1_ref[...] = h[:, HH:2 * HH]
    h2_ref[...] = h[:, 2 * HH:]


def _phi_stage(x, wcat, bcat):
    return pl.pallas_call(
        _phi_body,
        grid=(N // BLK,),
        in_specs=[pl.BlockSpec((BLK, D), lambda i: (i, 0)),
                  pl.BlockSpec((D, 3 * HH), lambda i: (0, 0)),
                  pl.BlockSpec((1, 3 * HH), lambda i: (0, 0))],
        out_specs=[pl.BlockSpec((BLK, HH), lambda i: (i, 0))] * 3,
        out_shape=[jax.ShapeDtypeStruct((N, HH), jnp.float32)] * 3,
    )(x, wcat, bcat)


def _sc_pass_a(h1, h2, src_a, dst_a, zrow, zcnt):
    mesh = plsc.VectorSubcoreMesh(core_axis_name="c", subcore_axis_name="s",
                                  num_cores=NC, num_subcores=NS)

    @functools.partial(
        pl.kernel, mesh=mesh,
        out_type=[jax.ShapeDtypeStruct((NC, N_PAD, HH), jnp.float32),
                  jax.ShapeDtypeStruct((N_PAD,), jnp.float32)],
        scratch_types=[
            pltpu.VMEM_SHARED((N_PAD, HH), jnp.float32),
            pltpu.VMEM_SHARED((N_PAD,), jnp.float32),
            pltpu.VMEM((WAVES_A, B), jnp.int32),
            pltpu.VMEM((WAVES_A, B), jnp.int32),
            pltpu.VMEM((B, HH), jnp.float32),
            pltpu.VMEM((B,), jnp.float32),
            pltpu.SemaphoreType.DMA,
        ])
    def k(h1_hbm, h2_hbm, src_hbm, dst_hbm, zrow_hbm, zcnt_hbm,
          s_out, cnt_out, acc, cacc, src_v, dst_v, rows_v, ones_v, sem):
        c = lax.axis_index("c")
        s = lax.axis_index("s")
        base = s * RT
        for t in range(RT // B):
            pltpu.sync_copy(zrow_hbm, acc.at[pl.ds(base + t * B, B)])
        pltpu.sync_copy(zcnt_hbm, cacc.at[pl.ds(base, RT)])
        for t in range(B // LANES):
            ones_v[pl.ds(t * LANES, LANES)] = jnp.ones((LANES,), jnp.float32)
        pltpu.sync_copy(src_hbm.at[s], src_v)
        pltpu.sync_copy(dst_hbm.at[s], dst_v)
        plsc.subcore_barrier()

        @pl.when(c == 0)
        def _():
            def body(j, carry):
                pltpu.async_copy(h1_hbm.at[src_v.at[j]], rows_v, sem).wait()
                pltpu.sync_copy(rows_v, acc.at[dst_v.at[j]], add=True)
                pltpu.sync_copy(ones_v, cacc.at[dst_v.at[j]], add=True)
                return carry
            lax.fori_loop(0, WAVES_A, body, 0)

        @pl.when(c == 1)
        def _():
            def body(j, carry):
                pltpu.async_copy(h2_hbm.at[src_v.at[j]], rows_v, sem).wait()
                pltpu.sync_copy(rows_v, acc.at[dst_v.at[j]], add=True)
                return carry
            lax.fori_loop(0, WAVES_A, body, 0)

        plsc.subcore_barrier()
        pltpu.sync_copy(acc.at[pl.ds(base, RT)], s_out.at[c, pl.ds(base, RT)])

        @pl.when(c == 0)
        def _():
            pltpu.sync_copy(cacc.at[pl.ds(base, RT)],
                            cnt_out.at[pl.ds(base, RT)])

    return k(h1, h2, src_a, dst_a, zrow, zcnt)


def _sc_pass_b(p, cnt, src_b, dst_b, zrow, zcnt):
    mesh = plsc.VectorSubcoreMesh(core_axis_name="c", subcore_axis_name="s",
                                  num_cores=NC, num_subcores=NS)

    @functools.partial(
        pl.kernel, mesh=mesh,
        out_type=[jax.ShapeDtypeStruct((NC, N_PAD, HH), jnp.float32),
                  jax.ShapeDtypeStruct((NC, N_PAD), jnp.float32)],
        scratch_types=[
            pltpu.VMEM_SHARED((N_PAD, HH), jnp.float32),
            pltpu.VMEM_SHARED((N_PAD,), jnp.float32),
            pltpu.VMEM((WAVES_B, B), jnp.int32),
            pltpu.VMEM((WAVES_B, B), jnp.int32),
            pltpu.VMEM((B, HH), jnp.float32),
            pltpu.VMEM((B,), jnp.float32),
            pltpu.SemaphoreType.DMA,
            pltpu.SemaphoreType.DMA,
        ])
    def k(p_hbm, cnt_hbm, src_hbm, dst_hbm, zrow_hbm, zcnt_hbm,
          sp_out, q_out, acc, cacc, src_v, dst_v, rows_v, cval_v, sem, sem2):
        c = lax.axis_index("c")
        s = lax.axis_index("s")
        w = c * NS + s
        base = s * RT
        for t in range(RT // B):
            pltpu.sync_copy(zrow_hbm, acc.at[pl.ds(base + t * B, B)])
        pltpu.sync_copy(zcnt_hbm, cacc.at[pl.ds(base, RT)])
        pltpu.sync_copy(src_hbm.at[w], src_v)
        pltpu.sync_copy(dst_hbm.at[w], dst_v)
        plsc.subcore_barrier()

        def body(j, carry):
            cg = pltpu.async_copy(cnt_hbm.at[src_v.at[j]], cval_v, sem2)
            pltpu.async_copy(p_hbm.at[src_v.at[j]], rows_v, sem).wait()
            cg.wait()
            pltpu.sync_copy(rows_v, acc.at[dst_v.at[j]], add=True)
            pltpu.sync_copy(cval_v, cacc.at[dst_v.at[j]], add=True)
            return carry
        lax.fori_loop(0, WAVES_B, body, 0)

        plsc.subcore_barrier()
        pltpu.sync_copy(acc.at[pl.ds(base, RT)], sp_out.at[c, pl.ds(base, RT)])
        pltpu.sync_copy(cacc.at[pl.ds(base, RT)],
                        q_out.at[c, pl.ds(base, RT)])

    return k(p, cnt, src_b, dst_b, zrow, zcnt)


def _mid_body(h0_ref, h1_ref, s1_ref, s2_ref, h2_ref, cnt_ref,
              rho0_ref, rho0b_ref, rho1_ref, rho1b_ref,
              n0_ref, n1_ref, p_ref):
    deg = cnt_ref[...] + 1.0
    h0 = h0_ref[...]
    n0_ref[...] = _relu(
        jnp.dot(jnp.concatenate([h0, h0], axis=1), rho0_ref[...],
                preferred_element_type=jnp.float32) + rho0b_ref[...])
    h1 = h1_ref[...]
    agg1 = (s1_ref[0] + h1) / deg
    n1_ref[...] = _relu(
        jnp.dot(jnp.concatenate([agg1, h1], axis=1), rho1_ref[...],
                preferred_element_type=jnp.float32) + rho1b_ref[...])
    p_ref[...] = s2_ref[0] + h2_ref[...]


def _mid_stage(h0, h1, s_out, h2, cnt2d, rho0_W, rho0_b, rho1_W, rho1_b):
    return pl.pallas_call(
        _mid_body,
        grid=(N // BLK,),
        in_specs=[pl.BlockSpec((BLK, HH), lambda i: (i, 0)),
                  pl.BlockSpec((BLK, HH), lambda i: (i, 0)),
                  pl.BlockSpec((1, BLK, HH), lambda i: (0, i, 0)),
                  pl.BlockSpec((1, BLK, HH), lambda i: (1, i, 0)),
                  pl.BlockSpec((BLK, HH), lambda i: (i, 0)),
                  pl.BlockSpec((BLK, 1), lambda i: (i, 0)),
                  pl.BlockSpec((2 * HH, 2 * HH), lambda i: (0, 0)),
                  pl.BlockSpec((1, 2 * HH), lambda i: (0, 0)),
                  pl.BlockSpec((2 * HH, 2 * HH), lambda i: (0, 0)),
                  pl.BlockSpec((1, 2 * HH), lambda i: (0, 0))],
        out_specs=[pl.BlockSpec((BLK, 2 * HH), lambda i: (i, 0)),
                   pl.BlockSpec((BLK, 2 * HH), lambda i: (i, 0)),
                   pl.BlockSpec((BLK, HH), lambda i: (i, 0))],
        out_shape=[jax.ShapeDtypeStruct((N, 2 * HH), jnp.float32),
                   jax.ShapeDtypeStruct((N, 2 * HH), jnp.float32),
                   jax.ShapeDtypeStruct((N, HH), jnp.float32)],
    )(h0, h1, s_out, s_out, h2, cnt2d, rho0_W, rho0_b, rho1_W, rho1_b)


def _out_body(sp_ref, q_ref, cnt_ref, p_ref, h2_ref, n0_ref, n1_ref,
              rho2_ref, rho2b_ref, rop_ref, ropb_ref, ror_ref, rorb_ref,
              out_ref):
    cnt = cnt_ref[...]
    pdeg = q_ref[0, 0] + q_ref[1, 0] + 2.0 * cnt + 1.0
    agg2 = (sp_ref[0] + sp_ref[1] + p_ref[...]) / pdeg
    n2 = _relu(
        jnp.dot(jnp.concatenate([agg2, h2_ref[...]], axis=1), rho2_ref[...],
                preferred_element_type=jnp.float32) + rho2b_ref[...])
    rop = rop_ref[...]
    ropb = ropb_ref[...]
    hro0 = _relu(jnp.dot(n0_ref[...], rop,
                         preferred_element_type=jnp.float32) + ropb)
    hro1 = _relu(jnp.dot(n1_ref[...], rop,
                         preferred_element_type=jnp.float32) + ropb)
    hro2 = _relu(jnp.dot(n2, rop,
                         preferred_element_type=jnp.float32) + ropb)
    pooled = (hro0 + hro1 + hro2) * (1.0 / 3.0)
    out_ref[...] = _relu(jnp.dot(pooled, ror_ref[...],
                                 preferred_element_type=jnp.float32)
                         + rorb_ref[...])


def _out_stage(sp, q2, cnt2d, p, h2, n0, n1,
               rho2_W, rho2_b, ro_phi_W, ro_phi_b, ro_rho_W, ro_rho_b):
    hfull = 2 * HH
    return pl.pallas_call(
        _out_body,
        grid=(N // BLK,),
        in_specs=[pl.BlockSpec((2, BLK, HH), lambda i: (0, i, 0)),
                  pl.BlockSpec((2, 1, BLK, 1), lambda i: (0, 0, i, 0)),
                  pl.BlockSpec((BLK, 1), lambda i: (i, 0)),
                  pl.BlockSpec((BLK, HH), lambda i: (i, 0)),
                  pl.BlockSpec((BLK, HH), lambda i: (i, 0)),
                  pl.BlockSpec((BLK, hfull), lambda i: (i, 0)),
                  pl.BlockSpec((BLK, hfull), lambda i: (i, 0)),
                  pl.BlockSpec((hfull, hfull), lambda i: (0, 0)),
                  pl.BlockSpec((1, hfull), lambda i: (0, 0)),
                  pl.BlockSpec((hfull, hfull), lambda i: (0, 0)),
                  pl.BlockSpec((1, hfull), lambda i: (0, 0)),
                  pl.BlockSpec((hfull, hfull), lambda i: (0, 0)),
                  pl.BlockSpec((1, hfull), lambda i: (0, 0))],
        out_specs=pl.BlockSpec((BLK, hfull), lambda i: (i, 0)),
        out_shape=jax.ShapeDtypeStruct((N, hfull), jnp.float32),
    )(sp, q2, cnt2d, p, h2, n0, n1,
      rho2_W, rho2_b, ro_phi_W, ro_phi_b, ro_rho_W, ro_rho_b)


def kernel(x, edge_index,
           phi0_W, phi0_b, rho0_W, rho0_b,
           phi1_W, phi1_b, rho1_W, rho1_b,
           phi2_W, phi2_b, rho2_W, rho2_b,
           ro_phi_W, ro_phi_b, ro_rho_W, ro_rho_b):
    f32 = jnp.float32
    ei = edge_index.astype(jnp.int32)
    pad = E_PAD - E
    src_p = jnp.concatenate([ei[0], jnp.zeros((pad,), jnp.int32)])
    dst_p = jnp.concatenate([ei[1], jnp.full((pad,), N, jnp.int32)])
    src_a = src_p.reshape(NS, WAVES_A, B)
    dst_a = dst_p.reshape(NS, WAVES_A, B)
    src_b = src_p.reshape(NC * NS, WAVES_B, B)
    dst_b = dst_p.reshape(NC * NS, WAVES_B, B)
    zrow = jnp.zeros((B, HH), f32)
    zcnt = jnp.zeros((RT,), f32)

    wcat = jnp.concatenate([phi0_W, phi1_W, phi2_W], axis=1)
    bcat = jnp.concatenate([phi0_b, phi1_b, phi2_b]).reshape(1, 3 * HH)

    h0, h1, h2 = _phi_stage(x, wcat, bcat)
    s_out, cnt = _sc_pass_a(h1, h2, src_a, dst_a, zrow, zcnt)
    cnt2d = cnt[:N].reshape(N, 1)
    n0, n1, p = _mid_stage(h0, h1, s_out, h2, cnt2d,
                           rho0_W, rho0_b.reshape(1, -1),
                           rho1_W, rho1_b.reshape(1, -1))
    sp, q = _sc_pass_b(p, cnt, src_b, dst_b, zrow, zcnt)
    q2 = q[:, :N].reshape(NC, 1, N, 1)
    out = _out_stage(sp, q2, cnt2d, p, h2, n0, n1,
                     rho2_W, rho2_b.reshape(1, -1),
                     ro_phi_W, ro_phi_b.reshape(1, -1),
                     ro_rho_W, ro_rho_b.reshape(1, -1))
    return out


# speedup vs baseline: 5.5596x; 1.1387x over previous
import functools

import jax
import jax.numpy as jnp
from jax import lax
from jax.experimental import pallas as pl
from jax.experimental.pallas import tpu as pltpu
from jax.experimental.pallas import tpu_sc as plsc

N = 10000
D = 256
HH = 128
E = 160000
NC, NS, LANES = 2, 16, 16
N_PAD = 10240
E_PAD = 163840
B = 128
WAVES_A = E_PAD // (NS * B)
WAVES_B = E_PAD // (NC * NS * B)
RT = N_PAD // NS
BLK = 2000


def _relu(v):
    return jnp.maximum(v, 0.0)


def _phi_body(x_ref, w_ref, b_ref, h0_ref, h1_ref, h2_ref):
    h = _relu(jnp.dot(x_ref[...], w_ref[...],
                      preferred_element_type=jnp.float32) + b_ref[...])
    h0_ref[...] = h[:, :HH]
    h1_ref[...] = h[:, HH:2 * HH]
    h2_ref[...] = h[:, 2 * HH:]


def _phi_stage(x, wcat, bcat):
    return pl.pallas_call(
        _phi_body,
        grid=(N // BLK,),
        in_specs=[pl.BlockSpec((BLK, D), lambda i: (i, 0)),
                  pl.BlockSpec((D, 3 * HH), lambda i: (0, 0)),
                  pl.BlockSpec((1, 3 * HH), lambda i: (0, 0))],
        out_specs=[pl.BlockSpec((BLK, HH), lambda i: (i, 0))] * 3,
        out_shape=[jax.ShapeDtypeStruct((N, HH), jnp.float32)] * 3,
    )(x, wcat, bcat)


NBUF = 2
CH = 8


def _sc_pass_a(h1, h2, src_a, dst_a, zrow, zcnt):
    mesh = plsc.VectorSubcoreMesh(core_axis_name="c", subcore_axis_name="s",
                                  num_cores=NC, num_subcores=NS)
    kch = WAVES_A // CH

    @functools.partial(
        pl.kernel, mesh=mesh,
        out_type=[jax.ShapeDtypeStruct((NC, N_PAD, HH), jnp.float32),
                  jax.ShapeDtypeStruct((N_PAD,), jnp.float32)],
        scratch_types=[
            pltpu.VMEM_SHARED((N_PAD, HH), jnp.float32),
            pltpu.VMEM_SHARED((N_PAD,), jnp.float32),
            pltpu.VMEM((2, CH, B), jnp.int32),
            pltpu.VMEM((WAVES_A, B), jnp.int32),
            pltpu.VMEM((NBUF, B, HH), jnp.float32),
            pltpu.VMEM((B,), jnp.float32),
        ] + [pltpu.SemaphoreType.DMA] * 6)
    def k(h1_hbm, h2_hbm, src_hbm, dst_hbm, zrow_hbm, zcnt_hbm,
          s_out, cnt_out, acc, cacc, srcch, dst_v, rows_v, ones_v, *sems):
        sem_g = sems[0:2]
        sem_s = sems[2:4]
        sem_i = sems[4:6]
        c = lax.axis_index("c")
        s = lax.axis_index("s")
        base = s * RT
        for t in range(RT // B):
            pltpu.sync_copy(zrow_hbm, acc.at[pl.ds(base + t * B, B)])
        pltpu.sync_copy(zcnt_hbm, cacc.at[pl.ds(base, RT)])
        for t in range(B // LANES):
            ones_v[pl.ds(t * LANES, LANES)] = jnp.ones((LANES,), jnp.float32)
        pltpu.sync_copy(dst_hbm.at[s], dst_v)
        pltpu.sync_copy(src_hbm.at[s, pl.ds(0, CH)], srcch.at[0])
        plsc.subcore_barrier()

        def run(h_hbm, count):
            pltpu.async_copy(h_hbm.at[srcch.at[0, 0]], rows_v.at[0], sem_g[0])
            pltpu.async_copy(src_hbm.at[s, pl.ds(CH, CH)], srcch.at[1],
                             sem_i[1])

            def body(t_, carry):
                j0 = t_ * 2 * CH
                for half in range(2):
                    for i in range(CH):
                        j = j0 + half * CH + i
                        b = i % 2
                        bn = 1 - b

                        @pl.when(j + 1 < WAVES_A)
                        def _():
                            if i == CH - 1:
                                pltpu.make_async_copy(
                                    src_hbm.at[s, pl.ds(0, CH)],
                                    srcch.at[1 - half],
                                    sem_i[1 - half]).wait()
                            @pl.when(j + 1 >= NBUF)
                            def _():
                                pltpu.make_async_copy(
                                    zrow_hbm, rows_v.at[bn], sem_s[bn]).wait()
                            if i == CH - 1:
                                nxt = srcch.at[1 - half, 0]
                            else:
                                nxt = srcch.at[half, i + 1]
                            pltpu.async_copy(h_hbm.at[nxt], rows_v.at[bn],
                                             sem_g[bn])

                        pltpu.make_async_copy(zrow_hbm, rows_v.at[b],
                                              sem_g[b]).wait()
                        pltpu.async_copy(rows_v.at[b], acc.at[dst_v.at[j]],
                                         sem_s[b], add=True)
                        if count:
                            pltpu.sync_copy(ones_v, cacc.at[dst_v.at[j]],
                                            add=True)
                        if i == CH - 1:
                            q = t_ * 2 + half

                            @pl.when(q + 2 < kch)
                            def _():
                                pltpu.async_copy(
                                    src_hbm.at[s, pl.ds((q + 2) * CH, CH)],
                                    srcch.at[half], sem_i[half])
                return carry

            lax.fori_loop(0, WAVES_A // (2 * CH), body, 0)
            for b in range(NBUF):
                pltpu.make_async_copy(zrow_hbm, rows_v.at[b], sem_s[b]).wait()

        @pl.when(c == 0)
        def _():
            run(h1_hbm, True)

        @pl.when(c == 1)
        def _():
            run(h2_hbm, False)

        plsc.subcore_barrier()
        pltpu.sync_copy(acc.at[pl.ds(base, RT)], s_out.at[c, pl.ds(base, RT)])

        @pl.when(c == 0)
        def _():
            pltpu.sync_copy(cacc.at[pl.ds(base, RT)],
                            cnt_out.at[pl.ds(base, RT)])

    return k(h1, h2, src_a, dst_a, zrow, zcnt)


def _sc_pass_b(p, cnt, src_b, dst_b, zrow, zcnt):
    mesh = plsc.VectorSubcoreMesh(core_axis_name="c", subcore_axis_name="s",
                                  num_cores=NC, num_subcores=NS)

    @functools.partial(
        pl.kernel, mesh=mesh,
        out_type=[jax.ShapeDtypeStruct((NC, N_PAD, HH), jnp.float32),
                  jax.ShapeDtypeStruct((NC, N_PAD), jnp.float32)],
        scratch_types=[
            pltpu.VMEM_SHARED((N_PAD, HH), jnp.float32),
            pltpu.VMEM_SHARED((N_PAD,), jnp.float32),
            pltpu.VMEM((WAVES_B, B), jnp.int32),
            pltpu.VMEM((WAVES_B, B), jnp.int32),
            pltpu.VMEM((NBUF, B, HH), jnp.float32),
            pltpu.VMEM((NBUF, B), jnp.float32),
        ] + [pltpu.SemaphoreType.DMA] * 6)
    def k(p_hbm, cnt_hbm, src_hbm, dst_hbm, zrow_hbm, zcnt_hbm,
          sp_out, q_out, acc, cacc, src_v, dst_v, rows_v, cval_v, *sems):
        sem_g = sems[0:2]
        sem_s = sems[2:4]
        sem_c = sems[4:6]
        c = lax.axis_index("c")
        s = lax.axis_index("s")
        w = c * NS + s
        base = s * RT
        for t in range(RT // B):
            pltpu.sync_copy(zrow_hbm, acc.at[pl.ds(base + t * B, B)])
        pltpu.sync_copy(zcnt_hbm, cacc.at[pl.ds(base, RT)])
        pltpu.sync_copy(src_hbm.at[w], src_v)
        pltpu.sync_copy(dst_hbm.at[w], dst_v)
        plsc.subcore_barrier()

        zc_dummy = zcnt_hbm.at[pl.ds(0, B)]
        pltpu.async_copy(p_hbm.at[src_v.at[0]], rows_v.at[0], sem_g[0])
        pltpu.async_copy(cnt_hbm.at[src_v.at[0]], cval_v.at[0], sem_c[0])

        def body(k_, carry):
            for b in range(NBUF):
                j = k_ * NBUF + b
                bn = 1 - b

                @pl.when(j + 1 < WAVES_B)
                def _():
                    @pl.when(j + 1 >= NBUF)
                    def _():
                        pltpu.make_async_copy(zrow_hbm, rows_v.at[bn],
                                              sem_s[bn]).wait()
                    pltpu.async_copy(p_hbm.at[src_v.at[j + 1]], rows_v.at[bn],
                                     sem_g[bn])
                    pltpu.async_copy(cnt_hbm.at[src_v.at[j + 1]],
                                     cval_v.at[bn], sem_c[bn])

                pltpu.make_async_copy(zrow_hbm, rows_v.at[b], sem_g[b]).wait()
                pltpu.async_copy(rows_v.at[b], acc.at[dst_v.at[j]], sem_s[b],
                                 add=True)
                pltpu.make_async_copy(zc_dummy, cval_v.at[b], sem_c[b]).wait()
                pltpu.sync_copy(cval_v.at[b], cacc.at[dst_v.at[j]], add=True)
            return carry

        lax.fori_loop(0, WAVES_B // NBUF, body, 0)
        for b in range(NBUF):
            pltpu.make_async_copy(zrow_hbm, rows_v.at[b], sem_s[b]).wait()

        plsc.subcore_barrier()
        pltpu.sync_copy(acc.at[pl.ds(base, RT)], sp_out.at[c, pl.ds(base, RT)])
        pltpu.sync_copy(cacc.at[pl.ds(base, RT)],
                        q_out.at[c, pl.ds(base, RT)])

    return k(p, cnt, src_b, dst_b, zrow, zcnt)


def _mid_body(h0_ref, h1_ref, s1_ref, s2_ref, h2_ref, cnt_ref,
              rho0_ref, rho0b_ref, rho1_ref, rho1b_ref,
              n0_ref, n1_ref, p_ref):
    deg = cnt_ref[...] + 1.0
    h0 = h0_ref[...]
    n0_ref[...] = _relu(
        jnp.dot(jnp.concatenate([h0, h0], axis=1), rho0_ref[...],
                preferred_element_type=jnp.float32) + rho0b_ref[...])
    h1 = h1_ref[...]
    agg1 = (s1_ref[0] + h1) / deg
    n1_ref[...] = _relu(
        jnp.dot(jnp.concatenate([agg1, h1], axis=1), rho1_ref[...],
                preferred_element_type=jnp.float32) + rho1b_ref[...])
    p_ref[...] = s2_ref[0] + h2_ref[...]


def _mid_stage(h0, h1, s_out, h2, cnt2d, rho0_W, rho0_b, rho1_W, rho1_b):
    return pl.pallas_call(
        _mid_body,
        grid=(N // BLK,),
        in_specs=[pl.BlockSpec((BLK, HH), lambda i: (i, 0)),
                  pl.BlockSpec((BLK, HH), lambda i: (i, 0)),
                  pl.BlockSpec((1, BLK, HH), lambda i: (0, i, 0)),
                  pl.BlockSpec((1, BLK, HH), lambda i: (1, i, 0)),
                  pl.BlockSpec((BLK, HH), lambda i: (i, 0)),
                  pl.BlockSpec((BLK, 1), lambda i: (i, 0)),
                  pl.BlockSpec((2 * HH, 2 * HH), lambda i: (0, 0)),
                  pl.BlockSpec((1, 2 * HH), lambda i: (0, 0)),
                  pl.BlockSpec((2 * HH, 2 * HH), lambda i: (0, 0)),
                  pl.BlockSpec((1, 2 * HH), lambda i: (0, 0))],
        out_specs=[pl.BlockSpec((BLK, 2 * HH), lambda i: (i, 0)),
                   pl.BlockSpec((BLK, 2 * HH), lambda i: (i, 0)),
                   pl.BlockSpec((BLK, HH), lambda i: (i, 0))],
        out_shape=[jax.ShapeDtypeStruct((N, 2 * HH), jnp.float32),
                   jax.ShapeDtypeStruct((N, 2 * HH), jnp.float32),
                   jax.ShapeDtypeStruct((N, HH), jnp.float32)],
    )(h0, h1, s_out, s_out, h2, cnt2d, rho0_W, rho0_b, rho1_W, rho1_b)


def _out_body(sp_ref, q_ref, cnt_ref, p_ref, h2_ref, n0_ref, n1_ref,
              rho2_ref, rho2b_ref, rop_ref, ropb_ref, ror_ref, rorb_ref,
              out_ref):
    cnt = cnt_ref[...]
    pdeg = q_ref[0, 0] + q_ref[1, 0] + 2.0 * cnt + 1.0
    agg2 = (sp_ref[0] + sp_ref[1] + p_ref[...]) / pdeg
    n2 = _relu(
        jnp.dot(jnp.concatenate([agg2, h2_ref[...]], axis=1), rho2_ref[...],
                preferred_element_type=jnp.float32) + rho2b_ref[...])
    rop = rop_ref[...]
    ropb = ropb_ref[...]
    hro0 = _relu(jnp.dot(n0_ref[...], rop,
                         preferred_element_type=jnp.float32) + ropb)
    hro1 = _relu(jnp.dot(n1_ref[...], rop,
                         preferred_element_type=jnp.float32) + ropb)
    hro2 = _relu(jnp.dot(n2, rop,
                         preferred_element_type=jnp.float32) + ropb)
    pooled = (hro0 + hro1 + hro2) * (1.0 / 3.0)
    out_ref[...] = _relu(jnp.dot(pooled, ror_ref[...],
                                 preferred_element_type=jnp.float32)
                         + rorb_ref[...])


def _out_stage(sp, q2, cnt2d, p, h2, n0, n1,
               rho2_W, rho2_b, ro_phi_W, ro_phi_b, ro_rho_W, ro_rho_b):
    hfull = 2 * HH
    return pl.pallas_call(
        _out_body,
        grid=(N // BLK,),
        in_specs=[pl.BlockSpec((2, BLK, HH), lambda i: (0, i, 0)),
                  pl.BlockSpec((2, 1, BLK, 1), lambda i: (0, 0, i, 0)),
                  pl.BlockSpec((BLK, 1), lambda i: (i, 0)),
                  pl.BlockSpec((BLK, HH), lambda i: (i, 0)),
                  pl.BlockSpec((BLK, HH), lambda i: (i, 0)),
                  pl.BlockSpec((BLK, hfull), lambda i: (i, 0)),
                  pl.BlockSpec((BLK, hfull), lambda i: (i, 0)),
                  pl.BlockSpec((hfull, hfull), lambda i: (0, 0)),
                  pl.BlockSpec((1, hfull), lambda i: (0, 0)),
                  pl.BlockSpec((hfull, hfull), lambda i: (0, 0)),
                  pl.BlockSpec((1, hfull), lambda i: (0, 0)),
                  pl.BlockSpec((hfull, hfull), lambda i: (0, 0)),
                  pl.BlockSpec((1, hfull), lambda i: (0, 0))],
        out_specs=pl.BlockSpec((BLK, hfull), lambda i: (i, 0)),
        out_shape=jax.ShapeDtypeStruct((N, hfull), jnp.float32),
    )(sp, q2, cnt2d, p, h2, n0, n1,
      rho2_W, rho2_b, ro_phi_W, ro_phi_b, ro_rho_W, ro_rho_b)


def kernel(x, edge_index,
           phi0_W, phi0_b, rho0_W, rho0_b,
           phi1_W, phi1_b, rho1_W, rho1_b,
           phi2_W, phi2_b, rho2_W, rho2_b,
           ro_phi_W, ro_phi_b, ro_rho_W, ro_rho_b):
    f32 = jnp.float32
    ei = edge_index.astype(jnp.int32)
    pad = E_PAD - E
    src_p = jnp.concatenate([ei[0], jnp.zeros((pad,), jnp.int32)])
    dst_p = jnp.concatenate([ei[1], jnp.full((pad,), N, jnp.int32)])
    src_a = src_p.reshape(NS, WAVES_A, B)
    dst_a = dst_p.reshape(NS, WAVES_A, B)
    src_b = src_p.reshape(NC * NS, WAVES_B, B)
    dst_b = dst_p.reshape(NC * NS, WAVES_B, B)
    zrow = jnp.zeros((B, HH), f32)
    zcnt = jnp.zeros((RT,), f32)

    wcat = jnp.concatenate([phi0_W, phi1_W, phi2_W], axis=1)
    bcat = jnp.concatenate([phi0_b, phi1_b, phi2_b]).reshape(1, 3 * HH)

    h0, h1, h2 = _phi_stage(x, wcat, bcat)
    s_out, cnt = _sc_pass_a(h1, h2, src_a, dst_a, zrow, zcnt)
    cnt2d = cnt[:N].reshape(N, 1)
    n0, n1, p = _mid_stage(h0, h1, s_out, h2, cnt2d,
                           rho0_W, rho0_b.reshape(1, -1),
                           rho1_W, rho1_b.reshape(1, -1))
    sp, q = _sc_pass_b(p, cnt, src_b, dst_b, zrow, zcnt)
    q2 = q[:, :N].reshape(NC, 1, N, 1)
    out = _out_stage(sp, q2, cnt2d, p, h2, n0, n1,
                     rho2_W, rho2_b.reshape(1, -1),
                     ro_phi_W, ro_phi_b.reshape(1, -1),
                     ro_rho_W, ro_rho_b.reshape(1, -1))
    return out


# speedup vs baseline: 6.2024x; 1.1156x over previous
import functools

import jax
import jax.numpy as jnp
from jax import lax
from jax.experimental import pallas as pl
from jax.experimental.pallas import tpu as pltpu
from jax.experimental.pallas import tpu_sc as plsc

N = 10000
D = 256
HH = 128
E = 160000
NC, NS, LANES = 2, 16, 16
N_PAD = 10240
E_PAD = 163840
B = 64
WAVES_A = E_PAD // (NS * B)
WAVES_B = E_PAD // (NC * NS * B)
RT = N_PAD // NS
BLK = 2000
NBUF = 4
PF = 2
CH = 8


def _relu(v):
    return jnp.maximum(v, 0.0)


def _phi_body(x_ref, w_ref, b_ref, h0_ref, h1_ref, h2_ref):
    h = _relu(jnp.dot(x_ref[...], w_ref[...],
                      preferred_element_type=jnp.float32) + b_ref[...])
    h0_ref[...] = h[:, :HH]
    h1_ref[...] = h[:, HH:2 * HH]
    h2_ref[...] = h[:, 2 * HH:]


def _phi_stage(x, wcat, bcat):
    return pl.pallas_call(
        _phi_body,
        grid=(N // BLK,),
        in_specs=[pl.BlockSpec((BLK, D), lambda i: (i, 0)),
                  pl.BlockSpec((D, 3 * HH), lambda i: (0, 0)),
                  pl.BlockSpec((1, 3 * HH), lambda i: (0, 0))],
        out_specs=[pl.BlockSpec((BLK, HH), lambda i: (i, 0))] * 3,
        out_shape=[jax.ShapeDtypeStruct((N, HH), jnp.float32),
                   jax.ShapeDtypeStruct((N_PAD, HH), jnp.float32),
                   jax.ShapeDtypeStruct((N_PAD, HH), jnp.float32)],
    )(x, wcat, bcat)


def _init_rows(src_hbm_rows, acc, base):
    for t in range(RT // 128):
        pltpu.sync_copy(src_hbm_rows.at[pl.ds(base + t * 128, 128)],
                        acc.at[pl.ds(base + t * 128, 128)])


def _seg_pipeline(src_slab, dst_slab, srcch, dstch, sem_i,
                  h_hbm, rows_v, sem_g, acc, sem_s, zrow_hbm, waves,
                  ones_v=None, cacc=None,
                  cnt_hbm=None, cval_v=None, sem_c=None, zc_dummy=None):
    kch = waves // CH
    pltpu.async_copy(src_slab.at[pl.ds(CH, CH)], srcch.at[1], sem_i[0])
    pltpu.async_copy(dst_slab.at[pl.ds(CH, CH)], dstch.at[1], sem_i[1])
    for b in range(PF):
        pltpu.async_copy(h_hbm.at[srcch.at[0, b]], rows_v.at[b], sem_g[b])
        if cnt_hbm is not None:
            pltpu.async_copy(cnt_hbm.at[srcch.at[0, b]], cval_v.at[b],
                             sem_c[b])

    def body(t_, carry):
        for half in range(2):
            for i in range(CH):
                j = t_ * 2 * CH + half * CH + i
                b = i % NBUF
                bn = (i + PF) % NBUF
                jn_i = i + PF

                @pl.when(j + PF < waves)
                def _():
                    if jn_i == CH:
                        pltpu.make_async_copy(src_slab.at[pl.ds(0, CH)],
                                              srcch.at[1 - half],
                                              sem_i[0]).wait()
                        pltpu.make_async_copy(dst_slab.at[pl.ds(0, CH)],
                                              dstch.at[1 - half],
                                              sem_i[1]).wait()

                    @pl.when(j + PF >= NBUF)
                    def _():
                        pltpu.make_async_copy(zrow_hbm, rows_v.at[bn],
                                              sem_s[bn]).wait()
                    if jn_i < CH:
                        nxt = srcch.at[half, jn_i]
                    else:
                        nxt = srcch.at[1 - half, jn_i - CH]
                    pltpu.async_copy(h_hbm.at[nxt], rows_v.at[bn], sem_g[bn])
                    if cnt_hbm is not None:
                        pltpu.async_copy(cnt_hbm.at[nxt], cval_v.at[bn],
                                         sem_c[bn])

                drow = dstch.at[half, i]
                pltpu.make_async_copy(zrow_hbm, rows_v.at[b], sem_g[b]).wait()
                pltpu.async_copy(rows_v.at[b], acc.at[drow], sem_s[b],
                                 add=True)
                if ones_v is not None:
                    pltpu.sync_copy(ones_v, cacc.at[drow], add=True)
                if cnt_hbm is not None:
                    pltpu.make_async_copy(zc_dummy, cval_v.at[b],
                                          sem_c[b]).wait()
                    pltpu.sync_copy(cval_v.at[b], cacc.at[drow], add=True)
                if i == CH - 1:
                    q = t_ * 2 + half

                    @pl.when(q + 2 < kch)
                    def _():
                        pltpu.async_copy(src_slab.at[pl.ds((q + 2) * CH, CH)],
                                         srcch.at[half], sem_i[0])
                        pltpu.async_copy(dst_slab.at[pl.ds((q + 2) * CH, CH)],
                                         dstch.at[half], sem_i[1])
        return carry

    lax.fori_loop(0, waves // (2 * CH), body, 0)
    for b in range(NBUF):
        pltpu.make_async_copy(zrow_hbm, rows_v.at[b], sem_s[b]).wait()


def _sc_pass_a(h1, h2, src_a, dst_a, zrow, zcnt):
    mesh = plsc.VectorSubcoreMesh(core_axis_name="c", subcore_axis_name="s",
                                  num_cores=NC, num_subcores=NS)

    @functools.partial(
        pl.kernel, mesh=mesh,
        out_type=[jax.ShapeDtypeStruct((N_PAD, HH), jnp.float32),
                  jax.ShapeDtypeStruct((N_PAD, HH), jnp.float32),
                  jax.ShapeDtypeStruct((N_PAD,), jnp.float32)],
        scratch_types=[
            pltpu.VMEM_SHARED((N_PAD, HH), jnp.float32),
            pltpu.VMEM_SHARED((N_PAD,), jnp.float32),
            pltpu.VMEM((2, CH, B), jnp.int32),
            pltpu.VMEM((2, CH, B), jnp.int32),
            pltpu.VMEM((NBUF, B, HH), jnp.float32),
            pltpu.VMEM((B,), jnp.float32),
        ] + [pltpu.SemaphoreType.DMA] * 10)
    def k(h1_hbm, h2_hbm, src_hbm, dst_hbm, zrow_hbm, zcnt_hbm,
          g1_out, p_out, cnt_out, acc, cacc, srcch, dstch, rows_v, ones_v,
          *sems):
        sem_g = sems[0:NBUF]
        sem_s = sems[NBUF:2 * NBUF]
        sem_i = sems[2 * NBUF:2 * NBUF + 2]
        c = lax.axis_index("c")
        s = lax.axis_index("s")
        base = s * RT
        src_slab = src_hbm.at[s]
        dst_slab = dst_hbm.at[s]

        @pl.when(c == 0)
        def _():
            _init_rows(h1_hbm, acc, base)

        @pl.when(c == 1)
        def _():
            _init_rows(h2_hbm, acc, base)

        pltpu.sync_copy(zcnt_hbm, cacc.at[pl.ds(base, RT)])
        for t in range(B // LANES):
            ones_v[pl.ds(t * LANES, LANES)] = jnp.ones((LANES,), jnp.float32)
        pltpu.sync_copy(src_slab.at[pl.ds(0, CH)], srcch.at[0])
        pltpu.sync_copy(dst_slab.at[pl.ds(0, CH)], dstch.at[0])
        plsc.subcore_barrier()

        @pl.when(c == 0)
        def _():
            _seg_pipeline(src_slab, dst_slab, srcch, dstch, sem_i,
                          h1_hbm, rows_v, sem_g, acc, sem_s, zrow_hbm,
                          WAVES_A, ones_v=ones_v, cacc=cacc)

        @pl.when(c == 1)
        def _():
            _seg_pipeline(src_slab, dst_slab, srcch, dstch, sem_i,
                          h2_hbm, rows_v, sem_g, acc, sem_s, zrow_hbm,
                          WAVES_A)

        plsc.subcore_barrier()

        @pl.when(c == 0)
        def _():
            pltpu.sync_copy(acc.at[pl.ds(base, RT)],
                            g1_out.at[pl.ds(base, RT)])
            pltpu.sync_copy(cacc.at[pl.ds(base, RT)],
                            cnt_out.at[pl.ds(base, RT)])

        @pl.when(c == 1)
        def _():
            pltpu.sync_copy(acc.at[pl.ds(base, RT)],
                            p_out.at[pl.ds(base, RT)])

    return k(h1, h2, src_a, dst_a, zrow, zcnt)


def _sc_pass_b(p, cnt, src_b, dst_b, zrow, zcnt):
    mesh = plsc.VectorSubcoreMesh(core_axis_name="c", subcore_axis_name="s",
                                  num_cores=NC, num_subcores=NS)

    @functools.partial(
        pl.kernel, mesh=mesh,
        out_type=[jax.ShapeDtypeStruct((NC, N_PAD, HH), jnp.float32),
                  jax.ShapeDtypeStruct((NC, N_PAD), jnp.float32)],
        scratch_types=[
            pltpu.VMEM_SHARED((N_PAD, HH), jnp.float32),
            pltpu.VMEM_SHARED((N_PAD,), jnp.float32),
            pltpu.VMEM((2, CH, B), jnp.int32),
            pltpu.VMEM((2, CH, B), jnp.int32),
            pltpu.VMEM((NBUF, B, HH), jnp.float32),
            pltpu.VMEM((NBUF, B), jnp.float32),
        ] + [pltpu.SemaphoreType.DMA] * 14)
    def k(p_hbm, cnt_hbm, src_hbm, dst_hbm, zrow_hbm, zcnt_hbm,
          sp_out, q_out, acc, cacc, srcch, dstch, rows_v, cval_v, *sems):
        sem_g = sems[0:NBUF]
        sem_s = sems[NBUF:2 * NBUF]
        sem_c = sems[2 * NBUF:3 * NBUF]
        sem_i = sems[3 * NBUF:3 * NBUF + 2]
        c = lax.axis_index("c")
        s = lax.axis_index("s")
        w = c * NS + s
        base = s * RT
        src_slab = src_hbm.at[w]
        dst_slab = dst_hbm.at[w]

        @pl.when(c == 0)
        def _():
            _init_rows(p_hbm, acc, base)
            pltpu.sync_copy(cnt_hbm.at[pl.ds(base, RT)],
                            cacc.at[pl.ds(base, RT)])

        @pl.when(c == 1)
        def _():
            for t in range(RT // B):
                pltpu.sync_copy(zrow_hbm, acc.at[pl.ds(base + t * B, B)])
            pltpu.sync_copy(zcnt_hbm, cacc.at[pl.ds(base, RT)])

        pltpu.sync_copy(src_slab.at[pl.ds(0, CH)], srcch.at[0])
        pltpu.sync_copy(dst_slab.at[pl.ds(0, CH)], dstch.at[0])
        plsc.subcore_barrier()

        _seg_pipeline(src_slab, dst_slab, srcch, dstch, sem_i,
                      p_hbm, rows_v, sem_g, acc, sem_s, zrow_hbm, WAVES_B,
                      cnt_hbm=cnt_hbm, cval_v=cval_v, sem_c=sem_c,
                      zc_dummy=zcnt_hbm.at[pl.ds(0, B)],
                      cacc=cacc)

        plsc.subcore_barrier()
        pltpu.sync_copy(acc.at[pl.ds(base, RT)], sp_out.at[c, pl.ds(base, RT)])
        pltpu.sync_copy(cacc.at[pl.ds(base, RT)],
                        q_out.at[c, pl.ds(base, RT)])

    return k(p, cnt, src_b, dst_b, zrow, zcnt)


def _mid_body(h0_ref, h1_ref, g1_ref, cnt_ref,
              rho0_ref, rho0b_ref, rho1_ref, rho1b_ref,
              n0_ref, n1_ref):
    deg = cnt_ref[...] + 1.0
    h0 = h0_ref[...]
    n0_ref[...] = _relu(
        jnp.dot(jnp.concatenate([h0, h0], axis=1), rho0_ref[...],
                preferred_element_type=jnp.float32) + rho0b_ref[...])
    agg1 = g1_ref[...] / deg
    n1_ref[...] = _relu(
        jnp.dot(jnp.concatenate([agg1, h1_ref[...]], axis=1), rho1_ref[...],
                preferred_element_type=jnp.float32) + rho1b_ref[...])


def _mid_stage(h0, h1, g1, cnt2d, rho0_W, rho0_b, rho1_W, rho1_b):
    return pl.pallas_call(
        _mid_body,
        grid=(N // BLK,),
        in_specs=[pl.BlockSpec((BLK, HH), lambda i: (i, 0)),
                  pl.BlockSpec((BLK, HH), lambda i: (i, 0)),
                  pl.BlockSpec((BLK, HH), lambda i: (i, 0)),
                  pl.BlockSpec((BLK, 1), lambda i: (i, 0)),
                  pl.BlockSpec((2 * HH, 2 * HH), lambda i: (0, 0)),
                  pl.BlockSpec((1, 2 * HH), lambda i: (0, 0)),
                  pl.BlockSpec((2 * HH, 2 * HH), lambda i: (0, 0)),
                  pl.BlockSpec((1, 2 * HH), lambda i: (0, 0))],
        out_specs=[pl.BlockSpec((BLK, 2 * HH), lambda i: (i, 0)),
                   pl.BlockSpec((BLK, 2 * HH), lambda i: (i, 0))],
        out_shape=[jax.ShapeDtypeStruct((N, 2 * HH), jnp.float32),
                   jax.ShapeDtypeStruct((N, 2 * HH), jnp.float32)],
    )(h0, h1, g1, cnt2d, rho0_W, rho0_b, rho1_W, rho1_b)


def _out_body(sp_ref, q_ref, cnt_ref, h2_ref, n0_ref, n1_ref,
              rho2_ref, rho2b_ref, rop_ref, ropb_ref, ror_ref, rorb_ref,
              out_ref):
    pdeg = q_ref[0, 0] + q_ref[1, 0] + cnt_ref[...] + 1.0
    agg2 = (sp_ref[0] + sp_ref[1]) / pdeg
    n2 = _relu(
        jnp.dot(jnp.concatenate([agg2, h2_ref[...]], axis=1), rho2_ref[...],
                preferred_element_type=jnp.float32) + rho2b_ref[...])
    rop = rop_ref[...]
    ropb = ropb_ref[...]
    hro0 = _relu(jnp.dot(n0_ref[...], rop,
                         preferred_element_type=jnp.float32) + ropb)
    hro1 = _relu(jnp.dot(n1_ref[...], rop,
                         preferred_element_type=jnp.float32) + ropb)
    hro2 = _relu(jnp.dot(n2, rop,
                         preferred_element_type=jnp.float32) + ropb)
    pooled = (hro0 + hro1 + hro2) * (1.0 / 3.0)
    out_ref[...] = _relu(jnp.dot(pooled, ror_ref[...],
                                 preferred_element_type=jnp.float32)
                         + rorb_ref[...])


def _out_stage(sp, q2, cnt2d, h2, n0, n1,
               rho2_W, rho2_b, ro_phi_W, ro_phi_b, ro_rho_W, ro_rho_b):
    hfull = 2 * HH
    return pl.pallas_call(
        _out_body,
        grid=(N // BLK,),
        in_specs=[pl.BlockSpec((2, BLK, HH), lambda i: (0, i, 0)),
                  pl.BlockSpec((2, 1, BLK, 1), lambda i: (0, 0, i, 0)),
                  pl.BlockSpec((BLK, 1), lambda i: (i, 0)),
                  pl.BlockSpec((BLK, HH), lambda i: (i, 0)),
                  pl.BlockSpec((BLK, hfull), lambda i: (i, 0)),
                  pl.BlockSpec((BLK, hfull), lambda i: (i, 0)),
                  pl.BlockSpec((hfull, hfull), lambda i: (0, 0)),
                  pl.BlockSpec((1, hfull), lambda i: (0, 0)),
                  pl.BlockSpec((hfull, hfull), lambda i: (0, 0)),
                  pl.BlockSpec((1, hfull), lambda i: (0, 0)),
                  pl.BlockSpec((hfull, hfull), lambda i: (0, 0)),
                  pl.BlockSpec((1, hfull), lambda i: (0, 0))],
        out_specs=pl.BlockSpec((BLK, hfull), lambda i: (i, 0)),
        out_shape=jax.ShapeDtypeStruct((N, hfull), jnp.float32),
    )(sp, q2, cnt2d, h2, n0, n1,
      rho2_W, rho2_b, ro_phi_W, ro_phi_b, ro_rho_W, ro_rho_b)


def kernel(x, edge_index,
           phi0_W, phi0_b, rho0_W, rho0_b,
           phi1_W, phi1_b, rho1_W, rho1_b,
           phi2_W, phi2_b, rho2_W, rho2_b,
           ro_phi_W, ro_phi_b, ro_rho_W, ro_rho_b):
    f32 = jnp.float32
    ei = edge_index.astype(jnp.int32)
    pad = E_PAD - E
    src_p = jnp.concatenate([ei[0], jnp.zeros((pad,), jnp.int32)])
    dst_p = jnp.concatenate([ei[1], jnp.full((pad,), N, jnp.int32)])
    src_a = src_p.reshape(NS, WAVES_A, B)
    dst_a = dst_p.reshape(NS, WAVES_A, B)
    src_b = src_p.reshape(NC * NS, WAVES_B, B)
    dst_b = dst_p.reshape(NC * NS, WAVES_B, B)
    zrow = jnp.zeros((B, HH), f32)
    zcnt = jnp.zeros((RT,), f32)

    wcat = jnp.concatenate([phi0_W, phi1_W, phi2_W], axis=1)
    bcat = jnp.concatenate([phi0_b, phi1_b, phi2_b]).reshape(1, 3 * HH)

    h0, h1, h2 = _phi_stage(x, wcat, bcat)
    g1, p, cnt = _sc_pass_a(h1, h2, src_a, dst_a, zrow, zcnt)
    cnt2d = cnt[:N].reshape(N, 1)
    n0, n1 = _mid_stage(h0, h1, g1[:N], cnt2d,
                        rho0_W, rho0_b.reshape(1, -1),
                        rho1_W, rho1_b.reshape(1, -1))
    sp, q = _sc_pass_b(p, cnt, src_b, dst_b, zrow, zcnt)
    q2 = q[:, :N].reshape(NC, 1, N, 1)
    out = _out_stage(sp, q2, cnt2d, h2[:N], n0, n1,
                     rho2_W, rho2_b.reshape(1, -1),
                     ro_phi_W, ro_phi_b.reshape(1, -1),
                     ro_rho_W, ro_rho_b.reshape(1, -1))
    return out
